# Initial kernel scaffold; baseline (speedup 1.0000x reference)
import functools

import jax
import jax.numpy as jnp
from jax import lax
from jax.experimental import pallas as pl
from jax.experimental.pallas import tpu as pltpu
from jax.experimental.pallas import tpu_sc as plsc

_E = 8
_K = 2
_BK = 128
_INV_SQRT2 = 0.7071067811865476


def _router_body(x_ref, wr_ref, idx_ref, w_ref):
    x = x_ref[...]
    wr = wr_ref[...]
    logits = lax.dot_general(x, wr, (((1,), (1,)), ((), ())),
                             preferred_element_type=jnp.float32)
    m = jnp.max(logits, axis=1, keepdims=True)
    p = jnp.exp(logits - m)
    p = p / jnp.sum(p, axis=1, keepdims=True)
    ii = lax.broadcasted_iota(jnp.int32, p.shape, 1)
    m1 = jnp.max(p, axis=1, keepdims=True)
    a1 = jnp.min(jnp.where(p == m1, ii, _E), axis=1, keepdims=True)
    p2 = jnp.where(ii == a1, -1.0, p)
    m2 = jnp.max(p2, axis=1, keepdims=True)
    a2 = jnp.min(jnp.where(p2 == m2, ii, _E), axis=1, keepdims=True)
    s = jnp.maximum(m1 + m2, 1e-12)
    idx_ref[...] = jnp.where(ii == 0, a1, jnp.where(ii == 1, a2, 0))
    w_ref[...] = jnp.where(ii == 0, m1 / s, jnp.where(ii == 1, m2 / s, 0.0))


def _router(tokens, Wr, interpret=False):
    T, D = tokens.shape
    TB = 1024
    return pl.pallas_call(
        _router_body,
        grid=(T // TB,),
        in_specs=[pl.BlockSpec((TB, D), lambda i: (i, 0)),
                  pl.BlockSpec((_E, D), lambda i: (0, 0))],
        out_specs=[pl.BlockSpec((TB, _E), lambda i: (i, 0)),
                   pl.BlockSpec((TB, _E), lambda i: (i, 0))],
        out_shape=[jax.ShapeDtypeStruct((T, _E), jnp.int32),
                   jax.ShapeDtypeStruct((T, _E), jnp.float32)],
        interpret=interpret,
    )(tokens, Wr)


def _ffn_body(te_ref, x_ref, w1_ref, b1_ref, w2_ref, b2_ref, ws_ref, y_ref):
    del te_ref
    x = x_ref[...]
    h = lax.dot_general(x, w1_ref[0], (((1,), (1,)), ((), ())),
                        preferred_element_type=jnp.float32)
    h = h + b1_ref[...]
    h = 0.5 * h * (1.0 + lax.erf(h * _INV_SQRT2))
    y = lax.dot_general(h, w2_ref[0], (((1,), (1,)), ((), ())),
                        preferred_element_type=jnp.float32)
    y = y + b2_ref[...]
    y_ref[...] = y * ws_ref[...]


def _grouped_ffn(tile_expert, x_sorted, W1, b1, W2, b2, ws, interpret=False):
    P, D = x_sorted.shape
    NT = P // _BK
    grid_spec = pltpu.PrefetchScalarGridSpec(
        num_scalar_prefetch=1,
        grid=(NT,),
        in_specs=[
            pl.BlockSpec((_BK, D), lambda i, te: (i, 0)),
            pl.BlockSpec((1, D, D), lambda i, te: (te[i], 0, 0)),
            pl.BlockSpec((1, D), lambda i, te: (te[i], 0)),
            pl.BlockSpec((1, D, D), lambda i, te: (te[i], 0, 0)),
            pl.BlockSpec((1, D), lambda i, te: (te[i], 0)),
            pl.BlockSpec((_BK, 1), lambda i, te: (i, 0)),
        ],
        out_specs=pl.BlockSpec((_BK, D), lambda i, te: (i, 0)),
    )
    return pl.pallas_call(
        _ffn_body,
        grid_spec=grid_spec,
        out_shape=jax.ShapeDtypeStruct((P, D), jnp.float32),
        interpret=interpret,
    )(tile_expert, x_sorted, W1, b1, W2, b2, ws)


def _dispatch(eidx, wtop, T, P):
    A = T * _K
    ea = eidx[:, :_K].reshape(-1)
    wa = wtop[:, :_K].reshape(-1)
    perm = jnp.argsort(ea)
    counts = jnp.sum(ea[:, None] == jnp.arange(_E)[None, :], axis=0)
    starts = (jnp.cumsum(counts) - counts).astype(jnp.int32)
    tiles = (counts + _BK - 1) // _BK
    cum_tiles = jnp.cumsum(tiles)
    pad_start = ((cum_tiles - tiles) * _BK).astype(jnp.int32)
    es = ea[perm]
    s = jnp.arange(A, dtype=jnp.int32)
    pos_sorted = pad_start[es] + (s - starts[es])
    row_token = jnp.zeros((P,), jnp.int32).at[pos_sorted].set(
        (perm // _K).astype(jnp.int32))
    ws = jnp.zeros((P,), jnp.float32).at[pos_sorted].set(wa[perm])
    posA = jnp.zeros((A,), jnp.int32).at[perm].set(pos_sorted)
    ti = jnp.arange(P // _BK, dtype=jnp.int32)
    tile_expert = jnp.minimum(
        jnp.sum(ti[:, None] >= cum_tiles[None, :], axis=1), _E - 1
    ).astype(jnp.int32)
    return row_token, ws, posA[0::_K], posA[1::_K], tile_expert


def kernel(input_embeddings, Wr, W1, b1, W2, b2):
    Bs, Ss, D = input_embeddings.shape
    T = Bs * Ss
    P = (T * _K // _BK + _E) * _BK
    tokens = input_embeddings.reshape(T, D)

    eidx, wtop = _router(tokens, Wr)
    row_token, ws, pos0, pos1, tile_expert = _dispatch(eidx, wtop, T, P)

    x_sorted = jnp.take(tokens, row_token, axis=0)
    y = _grouped_ffn(tile_expert, x_sorted, W1, b1, W2, b2, ws[:, None])
    out = tokens + jnp.take(y, pos0, axis=0) + jnp.take(y, pos1, axis=0)
    return out.reshape(Bs, Ss, D)


# speedup vs baseline: 1.2191x; 1.2191x over previous
import functools

import jax
import jax.numpy as jnp
from jax import lax
from jax.experimental import pallas as pl
from jax.experimental.pallas import tpu as pltpu
from jax.experimental.pallas import tpu_sc as plsc

_E = 8
_K = 2
_BK = 128
_INV_SQRT2 = 0.7071067811865476


def _router_body(x_ref, wr_ref, idx_ref, w_ref):
    x = x_ref[...]
    wr = wr_ref[...]
    logits = lax.dot_general(x, wr, (((1,), (1,)), ((), ())),
                             preferred_element_type=jnp.float32)
    m = jnp.max(logits, axis=1, keepdims=True)
    p = jnp.exp(logits - m)
    p = p / jnp.sum(p, axis=1, keepdims=True)
    ii = lax.broadcasted_iota(jnp.int32, p.shape, 1)
    m1 = jnp.max(p, axis=1, keepdims=True)
    a1 = jnp.min(jnp.where(p == m1, ii, _E), axis=1, keepdims=True)
    p2 = jnp.where(ii == a1, -1.0, p)
    m2 = jnp.max(p2, axis=1, keepdims=True)
    a2 = jnp.min(jnp.where(p2 == m2, ii, _E), axis=1, keepdims=True)
    s = jnp.maximum(m1 + m2, 1e-12)
    idx_ref[...] = jnp.where(ii == 0, a1, jnp.where(ii == 1, a2, 0))
    w_ref[...] = jnp.where(ii == 0, m1 / s, jnp.where(ii == 1, m2 / s, 0.0))


def _router(tokens, Wr, interpret=False):
    T, D = tokens.shape
    TB = 1024
    return pl.pallas_call(
        _router_body,
        grid=(T // TB,),
        in_specs=[pl.BlockSpec((TB, D), lambda i: (i, 0)),
                  pl.BlockSpec((_E, D), lambda i: (0, 0))],
        out_specs=[pl.BlockSpec((TB, _E), lambda i: (i, 0)),
                   pl.BlockSpec((TB, _E), lambda i: (i, 0))],
        out_shape=[jax.ShapeDtypeStruct((T, _E), jnp.int32),
                   jax.ShapeDtypeStruct((T, _E), jnp.float32)],
        interpret=interpret,
    )(tokens, Wr)


def _ffn_body(te_ref, x_ref, w1_ref, b1_ref, w2_ref, b2_ref, ws_ref, y_ref):
    del te_ref
    x = x_ref[...]
    h = lax.dot_general(x, w1_ref[0], (((1,), (1,)), ((), ())),
                        preferred_element_type=jnp.float32)
    h = h + b1_ref[0]
    h = 0.5 * h * (1.0 + lax.erf(h * _INV_SQRT2))
    y = lax.dot_general(h, w2_ref[0], (((1,), (1,)), ((), ())),
                        preferred_element_type=jnp.float32)
    y = y + b2_ref[0]
    y_ref[...] = y * ws_ref[...]


def _grouped_ffn(tile_expert, x_sorted, W1, b1, W2, b2, ws, interpret=False):
    P, D = x_sorted.shape
    NT = P // _BK
    grid_spec = pltpu.PrefetchScalarGridSpec(
        num_scalar_prefetch=1,
        grid=(NT,),
        in_specs=[
            pl.BlockSpec((_BK, D), lambda i, te: (i, 0)),
            pl.BlockSpec((1, D, D), lambda i, te: (te[i], 0, 0)),
            pl.BlockSpec((1, 1, D), lambda i, te: (te[i], 0, 0)),
            pl.BlockSpec((1, D, D), lambda i, te: (te[i], 0, 0)),
            pl.BlockSpec((1, 1, D), lambda i, te: (te[i], 0, 0)),
            pl.BlockSpec((_BK, 1), lambda i, te: (i, 0)),
        ],
        out_specs=pl.BlockSpec((_BK, D), lambda i, te: (i, 0)),
    )
    return pl.pallas_call(
        _ffn_body,
        grid_spec=grid_spec,
        out_shape=jax.ShapeDtypeStruct((P, D), jnp.float32),
        interpret=interpret,
    )(tile_expert, x_sorted, W1, b1[:, None, :], W2, b2[:, None, :], ws)


def _dispatch(eidx, wtop, T, P):
    A = T * _K
    ea = eidx[:, :_K].reshape(-1)
    wa = wtop[:, :_K].reshape(-1)
    perm = jnp.argsort(ea)
    counts = jnp.sum(ea[:, None] == jnp.arange(_E)[None, :], axis=0)
    starts = (jnp.cumsum(counts) - counts).astype(jnp.int32)
    tiles = (counts + _BK - 1) // _BK
    cum_tiles = jnp.cumsum(tiles)
    pad_start = ((cum_tiles - tiles) * _BK).astype(jnp.int32)
    es = ea[perm]
    s = jnp.arange(A, dtype=jnp.int32)
    pos_sorted = pad_start[es] + (s - starts[es])
    row_token = jnp.zeros((P,), jnp.int32).at[pos_sorted].set(
        (perm // _K).astype(jnp.int32))
    ws = jnp.zeros((P,), jnp.float32).at[pos_sorted].set(wa[perm])
    posA = jnp.zeros((A,), jnp.int32).at[perm].set(pos_sorted)
    ti = jnp.arange(P // _BK, dtype=jnp.int32)
    tile_expert = jnp.minimum(
        jnp.sum(ti[:, None] >= cum_tiles[None, :], axis=1), _E - 1
    ).astype(jnp.int32)
    return row_token, ws, posA[0::_K], posA[1::_K], tile_expert


def kernel(input_embeddings, Wr, W1, b1, W2, b2):
    Bs, Ss, D = input_embeddings.shape
    T = Bs * Ss
    P = (T * _K // _BK + _E) * _BK
    tokens = input_embeddings.reshape(T, D)

    eidx, wtop = _router(tokens, Wr)
    row_token, ws, pos0, pos1, tile_expert = _dispatch(eidx, wtop, T, P)

    x_sorted = jnp.take(tokens, row_token, axis=0)
    y = _grouped_ffn(tile_expert, x_sorted, W1, b1, W2, b2, ws[:, None])
    out = tokens + jnp.take(y, pos0, axis=0) + jnp.take(y, pos1, axis=0)
    return out.reshape(Bs, Ss, D)


# speedup vs baseline: 1.3639x; 1.1188x over previous
import functools

import jax
import jax.numpy as jnp
from jax import lax
from jax.experimental import pallas as pl
from jax.experimental.pallas import tpu as pltpu
from jax.experimental.pallas import tpu_sc as plsc

_E = 8
_K = 2
_BK = 128
_INV_SQRT2 = 0.7071067811865476


def _router_body(x_ref, wr_ref, idx_ref, w_ref):
    x = x_ref[...]
    wr = wr_ref[...]
    logits = lax.dot_general(x, wr, (((1,), (1,)), ((), ())),
                             preferred_element_type=jnp.float32)
    m = jnp.max(logits, axis=1, keepdims=True)
    p = jnp.exp(logits - m)
    p = p / jnp.sum(p, axis=1, keepdims=True)
    ii = lax.broadcasted_iota(jnp.int32, p.shape, 1)
    m1 = jnp.max(p, axis=1, keepdims=True)
    a1 = jnp.min(jnp.where(p == m1, ii, _E), axis=1, keepdims=True)
    p2 = jnp.where(ii == a1, -1.0, p)
    m2 = jnp.max(p2, axis=1, keepdims=True)
    a2 = jnp.min(jnp.where(p2 == m2, ii, _E), axis=1, keepdims=True)
    s = jnp.maximum(m1 + m2, 1e-12)
    idx_ref[...] = jnp.where(ii == 0, a1, jnp.where(ii == 1, a2, 0))
    w_ref[...] = jnp.where(ii == 0, m1 / s, jnp.where(ii == 1, m2 / s, 0.0))


def _router(tokens, Wr, interpret=False):
    T, D = tokens.shape
    TB = 1024
    return pl.pallas_call(
        _router_body,
        grid=(T // TB,),
        in_specs=[pl.BlockSpec((TB, D), lambda i: (i, 0)),
                  pl.BlockSpec((_E, D), lambda i: (0, 0))],
        out_specs=[pl.BlockSpec((TB, _E), lambda i: (i, 0)),
                   pl.BlockSpec((TB, _E), lambda i: (i, 0))],
        out_shape=[jax.ShapeDtypeStruct((T, _E), jnp.int32),
                   jax.ShapeDtypeStruct((T, _E), jnp.float32)],
        interpret=interpret,
    )(tokens, Wr)


def _ffn_body(te_ref, x_ref, w1_ref, b1_ref, w2_ref, b2_ref, ws_ref, y_ref):
    del te_ref
    x = x_ref[...]
    h = lax.dot_general(x, w1_ref[0], (((1,), (1,)), ((), ())),
                        preferred_element_type=jnp.float32)
    h = h + b1_ref[0]
    h = 0.5 * h * (1.0 + lax.erf(h * _INV_SQRT2))
    y = lax.dot_general(h, w2_ref[0], (((1,), (1,)), ((), ())),
                        preferred_element_type=jnp.float32)
    y = y + b2_ref[0]
    y_ref[...] = y * ws_ref[...]


def _grouped_ffn(tile_expert, x_sorted, W1, b1, W2, b2, ws, interpret=False):
    P, D = x_sorted.shape
    NT = P // _BK
    grid_spec = pltpu.PrefetchScalarGridSpec(
        num_scalar_prefetch=1,
        grid=(NT,),
        in_specs=[
            pl.BlockSpec((_BK, D), lambda i, te: (i, 0)),
            pl.BlockSpec((1, D, D), lambda i, te: (te[i], 0, 0)),
            pl.BlockSpec((1, 1, D), lambda i, te: (te[i], 0, 0)),
            pl.BlockSpec((1, D, D), lambda i, te: (te[i], 0, 0)),
            pl.BlockSpec((1, 1, D), lambda i, te: (te[i], 0, 0)),
            pl.BlockSpec((_BK, 1), lambda i, te: (i, 0)),
        ],
        out_specs=pl.BlockSpec((_BK, D), lambda i, te: (i, 0)),
    )
    return pl.pallas_call(
        _ffn_body,
        grid_spec=grid_spec,
        out_shape=jax.ShapeDtypeStruct((P, D), jnp.float32),
        interpret=interpret,
    )(tile_expert, x_sorted, W1, b1[:, None, :], W2, b2[:, None, :], ws)


def _dispatch(eidx, wtop, T, P):
    A = T * _K
    ea = eidx[:, :_K].reshape(-1)
    wa = wtop[:, :_K].reshape(-1)
    onehot = (ea[:, None] == jnp.arange(_E)[None, :]).astype(jnp.int32)
    csum = jnp.cumsum(onehot, axis=0)
    counts = csum[-1]
    rank = jnp.sum(onehot * csum, axis=1) - 1
    tiles = (counts + _BK - 1) // _BK
    cum_tiles = jnp.cumsum(tiles)
    pad_start = ((cum_tiles - tiles) * _BK).astype(jnp.int32)
    pos_a = jnp.sum(onehot * pad_start[None, :], axis=1) + rank
    row_token = jnp.zeros((P,), jnp.int32).at[pos_a].set(
        (jnp.arange(A, dtype=jnp.int32) // _K))
    ws = jnp.zeros((P,), jnp.float32).at[pos_a].set(wa)
    pos2 = pos_a.reshape(T, _K)
    ti = jnp.arange(P // _BK, dtype=jnp.int32)
    tile_expert = jnp.minimum(
        jnp.sum(ti[:, None] >= cum_tiles[None, :], axis=1), _E - 1
    ).astype(jnp.int32)
    return row_token, ws, pos2[:, 0], pos2[:, 1], tile_expert


def kernel(input_embeddings, Wr, W1, b1, W2, b2):
    Bs, Ss, D = input_embeddings.shape
    T = Bs * Ss
    P = (T * _K // _BK + _E) * _BK
    tokens = input_embeddings.reshape(T, D)

    eidx, wtop = _router(tokens, Wr)
    row_token, ws, pos0, pos1, tile_expert = _dispatch(eidx, wtop, T, P)

    x_sorted = jnp.take(tokens, row_token, axis=0)
    y = _grouped_ffn(tile_expert, x_sorted, W1, b1, W2, b2, ws[:, None])
    out = tokens + jnp.take(y, pos0, axis=0) + jnp.take(y, pos1, axis=0)
    return out.reshape(Bs, Ss, D)
